# baseline (speedup 1.0000x reference)
import jax
import jax.numpy as jnp
from jax.experimental import pallas as pl
from jax.experimental.pallas import tpu as pltpu

_N = 1000
_S = 32
_R = 40
_BIG = 2**30


def _body(u_ref, l_ref, x_ref, y_ref, best_ref, idx_ref):
    i = pl.program_id(0)

    @pl.when(i == 0)
    def _init():
        best_ref[...] = jnp.full((_S, 1), -jnp.inf, jnp.float32)
        idx_ref[...] = jnp.zeros((_S, 1), jnp.int32)

    u = u_ref[...]
    lg = l_ref[...]
    uc = jnp.clip(u, 1e-06, 1.0 - 1e-06)
    scores = lg[None, :, :] - jnp.log(-jnp.log(uc))

    m2 = jnp.max(scores, axis=2)
    m = jnp.max(m2, axis=1, keepdims=True)

    row = jax.lax.broadcasted_iota(jnp.int32, (_S, _R, _N), 1)
    col = jax.lax.broadcasted_iota(jnp.int32, (_S, _R, _N), 2)
    flat = (i * _R + row) * _N + col
    cand = jnp.where(scores == m[:, :, None], flat, _BIG)
    ci2 = jnp.min(cand, axis=2)
    ci = jnp.min(ci2, axis=1, keepdims=True)

    better = m > best_ref[...]
    best_ref[...] = jnp.where(better, m, best_ref[...])
    idx_ref[...] = jnp.where(better, ci, idx_ref[...])

    @pl.when(i == pl.num_programs(0) - 1)
    def _fin():
        fidx = idx_ref[...]
        x_ref[...] = fidx // _N
        y_ref[...] = fidx - (fidx // _N) * _N


def kernel(uniform, logits):
    u3 = uniform.reshape(_S, _N, _N)
    grid = _N // _R
    x2, y2 = pl.pallas_call(
        _body,
        grid=(grid,),
        in_specs=[
            pl.BlockSpec((_S, _R, _N), lambda i: (0, i, 0)),
            pl.BlockSpec((_R, _N), lambda i: (i, 0)),
        ],
        out_specs=[
            pl.BlockSpec((_S, 1), lambda i: (0, 0)),
            pl.BlockSpec((_S, 1), lambda i: (0, 0)),
        ],
        out_shape=[
            jax.ShapeDtypeStruct((_S, 1), jnp.int32),
            jax.ShapeDtypeStruct((_S, 1), jnp.int32),
        ],
        scratch_shapes=[
            pltpu.VMEM((_S, 1), jnp.float32),
            pltpu.VMEM((_S, 1), jnp.int32),
        ],
    )(u3, logits)
    return x2.reshape(_S), y2.reshape(_S)
